# baseline (speedup 1.0000x reference)
import jax
import jax.numpy as jnp
from jax.experimental import pallas as pl

N = 10000
D = 128
H = 32
BLOCK = 2000

_CONTRACT_LHS0_RHS1 = (((0,), (1,)), ((), ()))
_CONTRACT_ROWCOL = (((1,), (0,)), ((), ()))


def _cell_kernel(x_ref, h_ref, c_ref, w_ref, col_ref, out_ref, pk_ref):
    gT = jax.lax.dot_general(w_ref[0:D], x_ref[...], _CONTRACT_LHS0_RHS1,
                             preferred_element_type=jnp.float32)
    gT = gT + jax.lax.dot_general(w_ref[D:D + H], h_ref[...], _CONTRACT_LHS0_RHS1,
                                  preferred_element_type=jnp.float32)
    bias = col_ref[0:1, :].T
    wc2 = col_ref[1:2, :].T
    wco = col_ref[2:3, :].T[0:H]
    wlin = col_ref[3:4, 0:H]
    blin = col_ref[4:5, 0:1]
    grp = jax.lax.broadcasted_iota(jnp.int32, (4 * H, 1), 0) // H
    is_t = grp == 2
    alpha = jnp.where(is_t, 2.0, 1.0).astype(jnp.float32)
    beta = jnp.where(is_t, -1.0, 0.0).astype(jnp.float32)
    gT = gT + bias
    cT = c_ref[...].T
    z = jnp.zeros_like(cT)
    c4T = jnp.concatenate([cT, cT, z, z], axis=0)
    preT = gT + wc2 * c4T
    sT = jax.nn.sigmoid(preT * alpha)
    actT = sT * alpha + beta
    iT = actT[0 * H:1 * H]
    fT = actT[1 * H:2 * H]
    tT = actT[2 * H:3 * H]
    cnT = fT * cT + iT * tT
    opreT = preT[3 * H:4 * H] + wco * cnT
    packT = jnp.concatenate([opreT, 2.0 * cnT], axis=0)
    spT = jax.nn.sigmoid(packT)
    h0T = spT[0 * H:1 * H] * (2.0 * spT[1 * H:2 * H] - 1.0)
    pkT = jnp.concatenate([h0T, cnT], axis=0)
    pk = pkT.T
    hr = jnp.maximum(pk[:, 0:H], 0.0)
    out_ref[...] = jnp.sum(hr * wlin, axis=1, keepdims=True) + blin
    pk_ref[...] = pk


def kernel(x, edge_index, edge_weight, h, c,
           W_xi, b_xi, W_hi, b_hi, w_ci, b_i,
           W_xf, b_xf, W_hf, b_hf, w_cf, b_f,
           W_xc, b_xc, W_hc, b_hc, b_c,
           W_xo, b_xo, W_ho, b_ho, w_co, b_o,
           W_lin, b_lin):
    wx = jnp.concatenate([W_xi, W_xf, W_xc, W_xo], axis=1)
    wh = jnp.concatenate([W_hi, W_hf, W_hc, W_ho], axis=1)
    w = jnp.concatenate([wx, wh], axis=0)
    z3H = jnp.zeros((3 * H,), jnp.float32)
    col = jnp.concatenate([
        b_xi + b_hi + b_i, b_xf + b_hf + b_f,
        b_xc + b_hc + b_c, b_xo + b_ho + b_o,
        w_ci, w_cf, z3H[0:2 * H],
        w_co, z3H,
        W_lin[:, 0], z3H,
        jnp.broadcast_to(b_lin, (H,)), z3H,
    ]).reshape(5, 4 * H)

    grid = (N // BLOCK,)
    rowmap = lambda i: (i, 0)
    fixed = lambda i: (0, 0)
    out, pk = pl.pallas_call(
        _cell_kernel,
        grid=grid,
        in_specs=[
            pl.BlockSpec((BLOCK, D), rowmap),
            pl.BlockSpec((BLOCK, H), rowmap),
            pl.BlockSpec((BLOCK, H), rowmap),
            pl.BlockSpec((D + H, 4 * H), fixed),
            pl.BlockSpec((5, 4 * H), fixed),
        ],
        out_specs=[
            pl.BlockSpec((BLOCK, 1), rowmap),
            pl.BlockSpec((BLOCK, 2 * H), rowmap),
        ],
        out_shape=[
            jax.ShapeDtypeStruct((N, 1), jnp.float32),
            jax.ShapeDtypeStruct((N, 2 * H), jnp.float32),
        ],
    )(x, h, c, w, col)
    return (out, pk[:, 0:H], pk[:, H:2 * H])


# speedup vs baseline: 1.1149x; 1.1149x over previous
import jax
import jax.numpy as jnp
from jax.experimental import pallas as pl

N = 10000
D = 128
H = 32
BLOCK = 2000

_CONTRACT_LHS0_RHS1 = (((0,), (1,)), ((), ()))
_CONTRACT_ROWCOL = (((1,), (0,)), ((), ()))


def _cell_kernel(x_ref, h_ref, c_ref, w_ref, col_ref, out_ref, h0_ref, cn_ref):
    gT = jax.lax.dot_general(w_ref[0:D], x_ref[...], _CONTRACT_LHS0_RHS1,
                             preferred_element_type=jnp.float32)
    gT = gT + jax.lax.dot_general(w_ref[D:D + H], h_ref[...], _CONTRACT_LHS0_RHS1,
                                  preferred_element_type=jnp.float32)
    bias = col_ref[0:1, :].T
    wc2 = col_ref[1:2, :].T
    wco = col_ref[2:3, :].T[0:H]
    wlin = col_ref[3:4, 0:H]
    blin = col_ref[4:5, 0:1]
    grp = jax.lax.broadcasted_iota(jnp.int32, (4 * H, 1), 0) // H
    is_t = grp == 2
    alpha = jnp.where(is_t, 2.0, 1.0).astype(jnp.float32)
    beta = jnp.where(is_t, -1.0, 0.0).astype(jnp.float32)
    gT = gT + bias
    cT = c_ref[...].T
    z = jnp.zeros_like(cT)
    c4T = jnp.concatenate([cT, cT, z, z], axis=0)
    preT = gT + wc2 * c4T
    sT = jax.nn.sigmoid(preT * alpha)
    actT = sT * alpha + beta
    iT = actT[0 * H:1 * H]
    fT = actT[1 * H:2 * H]
    tT = actT[2 * H:3 * H]
    cnT = fT * cT + iT * tT
    opreT = preT[3 * H:4 * H] + wco * cnT
    packT = jnp.concatenate([opreT, 2.0 * cnT], axis=0)
    spT = jax.nn.sigmoid(packT)
    h0T = spT[0 * H:1 * H] * (2.0 * spT[1 * H:2 * H] - 1.0)
    h0 = h0T.T
    cn = cnT.T
    hr = jnp.maximum(h0, 0.0)
    out_ref[...] = jnp.sum(hr * wlin, axis=1, keepdims=True) + blin
    h0_ref[...] = h0
    cn_ref[...] = cn


def kernel(x, edge_index, edge_weight, h, c,
           W_xi, b_xi, W_hi, b_hi, w_ci, b_i,
           W_xf, b_xf, W_hf, b_hf, w_cf, b_f,
           W_xc, b_xc, W_hc, b_hc, b_c,
           W_xo, b_xo, W_ho, b_ho, w_co, b_o,
           W_lin, b_lin):
    wx = jnp.concatenate([W_xi, W_xf, W_xc, W_xo], axis=1)
    wh = jnp.concatenate([W_hi, W_hf, W_hc, W_ho], axis=1)
    w = jnp.concatenate([wx, wh], axis=0)
    z3H = jnp.zeros((3 * H,), jnp.float32)
    col = jnp.concatenate([
        b_xi + b_hi + b_i, b_xf + b_hf + b_f,
        b_xc + b_hc + b_c, b_xo + b_ho + b_o,
        w_ci, w_cf, z3H[0:2 * H],
        w_co, z3H,
        W_lin[:, 0], z3H,
        b_lin, z3H, z3H[0:H - 1],
    ]).reshape(5, 4 * H)

    grid = (N // BLOCK,)
    rowmap = lambda i: (i, 0)
    fixed = lambda i: (0, 0)
    out, h0, cn = pl.pallas_call(
        _cell_kernel,
        grid=grid,
        in_specs=[
            pl.BlockSpec((BLOCK, D), rowmap),
            pl.BlockSpec((BLOCK, H), rowmap),
            pl.BlockSpec((BLOCK, H), rowmap),
            pl.BlockSpec((D + H, 4 * H), fixed),
            pl.BlockSpec((5, 4 * H), fixed),
        ],
        out_specs=[
            pl.BlockSpec((BLOCK, 1), rowmap),
            pl.BlockSpec((BLOCK, H), rowmap),
            pl.BlockSpec((BLOCK, H), rowmap),
        ],
        out_shape=[
            jax.ShapeDtypeStruct((N, 1), jnp.float32),
            jax.ShapeDtypeStruct((N, H), jnp.float32),
            jax.ShapeDtypeStruct((N, H), jnp.float32),
        ],
    )(x, h, c, w, col)
    return (out, h0, cn)


# speedup vs baseline: 1.1496x; 1.0312x over previous
import jax
import jax.numpy as jnp
from jax.experimental import pallas as pl

N = 10000
D = 128
H = 32
BLOCK = 5000

_CONTRACT_LHS0_RHS1 = (((0,), (1,)), ((), ()))
_CONTRACT_ROWCOL = (((1,), (0,)), ((), ()))


def _cell_kernel(x_ref, h_ref, c_ref, w_ref, col_ref, out_ref, h0_ref, cn_ref):
    gT = jax.lax.dot_general(w_ref[0:D], x_ref[...], _CONTRACT_LHS0_RHS1,
                             preferred_element_type=jnp.float32)
    gT = gT + jax.lax.dot_general(w_ref[D:D + H], h_ref[...], _CONTRACT_LHS0_RHS1,
                                  preferred_element_type=jnp.float32)
    bias = col_ref[0:1, :].T
    wc2 = col_ref[1:2, :].T
    wco = col_ref[2:3, :].T[0:H]
    wlin = col_ref[3:4, 0:H]
    blin = col_ref[4:5, 0:1]
    grp = jax.lax.broadcasted_iota(jnp.int32, (4 * H, 1), 0) // H
    is_t = grp == 2
    alpha = jnp.where(is_t, 2.0, 1.0).astype(jnp.float32)
    beta = jnp.where(is_t, -1.0, 0.0).astype(jnp.float32)
    gT = gT + bias
    cT = c_ref[...].T
    z = jnp.zeros_like(cT)
    c4T = jnp.concatenate([cT, cT, z, z], axis=0)
    preT = gT + wc2 * c4T
    sT = jax.nn.sigmoid(preT * alpha)
    actT = sT * alpha + beta
    iT = actT[0 * H:1 * H]
    fT = actT[1 * H:2 * H]
    tT = actT[2 * H:3 * H]
    cnT = fT * cT + iT * tT
    opreT = preT[3 * H:4 * H] + wco * cnT
    packT = jnp.concatenate([opreT, 2.0 * cnT], axis=0)
    spT = jax.nn.sigmoid(packT)
    h0T = spT[0 * H:1 * H] * (2.0 * spT[1 * H:2 * H] - 1.0)
    h0 = h0T.T
    cn = cnT.T
    hr = jnp.maximum(h0, 0.0)
    out_ref[...] = jnp.sum(hr * wlin, axis=1, keepdims=True) + blin
    h0_ref[...] = h0
    cn_ref[...] = cn


def kernel(x, edge_index, edge_weight, h, c,
           W_xi, b_xi, W_hi, b_hi, w_ci, b_i,
           W_xf, b_xf, W_hf, b_hf, w_cf, b_f,
           W_xc, b_xc, W_hc, b_hc, b_c,
           W_xo, b_xo, W_ho, b_ho, w_co, b_o,
           W_lin, b_lin):
    wx = jnp.concatenate([W_xi, W_xf, W_xc, W_xo], axis=1)
    wh = jnp.concatenate([W_hi, W_hf, W_hc, W_ho], axis=1)
    w = jnp.concatenate([wx, wh], axis=0)
    z3H = jnp.zeros((3 * H,), jnp.float32)
    col = jnp.concatenate([
        b_xi + b_hi + b_i, b_xf + b_hf + b_f,
        b_xc + b_hc + b_c, b_xo + b_ho + b_o,
        w_ci, w_cf, z3H[0:2 * H],
        w_co, z3H,
        W_lin[:, 0], z3H,
        b_lin, z3H, z3H[0:H - 1],
    ]).reshape(5, 4 * H)

    grid = (N // BLOCK,)
    rowmap = lambda i: (i, 0)
    fixed = lambda i: (0, 0)
    out, h0, cn = pl.pallas_call(
        _cell_kernel,
        grid=grid,
        in_specs=[
            pl.BlockSpec((BLOCK, D), rowmap),
            pl.BlockSpec((BLOCK, H), rowmap),
            pl.BlockSpec((BLOCK, H), rowmap),
            pl.BlockSpec((D + H, 4 * H), fixed),
            pl.BlockSpec((5, 4 * H), fixed),
        ],
        out_specs=[
            pl.BlockSpec((BLOCK, 1), rowmap),
            pl.BlockSpec((BLOCK, H), rowmap),
            pl.BlockSpec((BLOCK, H), rowmap),
        ],
        out_shape=[
            jax.ShapeDtypeStruct((N, 1), jnp.float32),
            jax.ShapeDtypeStruct((N, H), jnp.float32),
            jax.ShapeDtypeStruct((N, H), jnp.float32),
        ],
    )(x, h, c, w, col)
    return (out, h0, cn)
